# Initial kernel scaffold; baseline (speedup 1.0000x reference)
import jax
import jax.numpy as jnp
from jax.experimental import pallas as pl


def kernel(x, W_gnn, b_gnn, W_path, b_path, W1, b1, W2, b2, edge_index, edge_pairs, path_nodes, path_lengths):
    raise NotImplementedError("write your pallas kernel here")


# speedup vs baseline: 1.1632x; 1.1632x over previous
import functools

import jax
import jax.numpy as jnp
from jax.experimental import pallas as pl
from jax.experimental.pallas import tpu as pltpu

N = 10000
NPAD = 10240
D = 256
H = 128
P = 4096
L = 10
E = 160000

BR = 1024
BP = 1024


def _gnn_body(a0, a1, deg, w, b, h_out):
    inv = 1.0 / jnp.maximum(deg[...], 1.0)
    x0 = a0[...] * inv
    x1 = a1[...] * inv
    acc = jnp.dot(x0, w[0:H, :], preferred_element_type=jnp.float32)
    acc = acc + jnp.dot(x1, w[H:D, :], preferred_element_type=jnp.float32)
    h = jnp.maximum(acc + b[...], 0.0)
    rows = pl.program_id(0) * BR + jax.lax.broadcasted_iota(jnp.int32, h.shape, 0)
    h_out[...] = jnp.where(rows < N, h, 0.0)


def _gnn_dense(a0, a1, deg, w, b):
    return pl.pallas_call(
        _gnn_body,
        grid=(NPAD // BR,),
        in_specs=[
            pl.BlockSpec((BR, H), lambda i: (i, 0)),
            pl.BlockSpec((BR, H), lambda i: (i, 0)),
            pl.BlockSpec((BR, 1), lambda i: (i, 0)),
            pl.BlockSpec((D, D), lambda i: (0, 0)),
            pl.BlockSpec((1, D), lambda i: (0, 0)),
        ],
        out_specs=pl.BlockSpec((BR, D), lambda i: (i, 0)),
        out_shape=jax.ShapeDtypeStruct((NPAD, D), jnp.float32),
    )(a0, a1, deg, w, b)


def _pred_body(p0, p1, hu0, hu1, hv0, hv1, lens, wp, bp, w1, b1, w2t, b2, out):
    il = 1.0 / lens[...]
    m0 = p0[...] * il
    m1 = p1[...] * il
    hp = jnp.dot(m0, wp[0:H, :], preferred_element_type=jnp.float32)
    hp = hp + jnp.dot(m1, wp[H:D, :], preferred_element_type=jnp.float32)
    hp = 2.0 * jnp.maximum(hp + bp[...], 0.0)
    t = jnp.dot(hu0[...], w1[0:H, :], preferred_element_type=jnp.float32)
    t = t + jnp.dot(hu1[...], w1[H:2 * H, :], preferred_element_type=jnp.float32)
    t = t + jnp.dot(hv0[...], w1[2 * H:3 * H, :], preferred_element_type=jnp.float32)
    t = t + jnp.dot(hv1[...], w1[3 * H:4 * H, :], preferred_element_type=jnp.float32)
    t = t + jnp.dot(hp, w1[4 * H:6 * H, :], preferred_element_type=jnp.float32)
    t = jnp.maximum(t + b1[...], 0.0)
    out[...] = jnp.sum(t * w2t[...], axis=1, keepdims=True) + b2[...]


def _predictor(p0, p1, hu0, hu1, hv0, hv1, lens, wp, bp, w1, b1, w2t, b2):
    return pl.pallas_call(
        _pred_body,
        grid=(P // BP,),
        in_specs=[
            pl.BlockSpec((BP, H), lambda i: (i, 0)),
            pl.BlockSpec((BP, H), lambda i: (i, 0)),
            pl.BlockSpec((BP, H), lambda i: (i, 0)),
            pl.BlockSpec((BP, H), lambda i: (i, 0)),
            pl.BlockSpec((BP, H), lambda i: (i, 0)),
            pl.BlockSpec((BP, H), lambda i: (i, 0)),
            pl.BlockSpec((BP, 1), lambda i: (i, 0)),
            pl.BlockSpec((D, D), lambda i: (0, 0)),
            pl.BlockSpec((1, D), lambda i: (0, 0)),
            pl.BlockSpec((3 * D, D), lambda i: (0, 0)),
            pl.BlockSpec((1, D), lambda i: (0, 0)),
            pl.BlockSpec((1, D), lambda i: (0, 0)),
            pl.BlockSpec((1, 1), lambda i: (0, 0)),
        ],
        out_specs=pl.BlockSpec((BP, 1), lambda i: (i, 0)),
        out_shape=jax.ShapeDtypeStruct((P, 1), jnp.float32),
    )(p0, p1, hu0, hu1, hv0, hv1, lens, wp, bp, w1, b1, w2t, b2)


def kernel(x, W_gnn, b_gnn, W_path, b_path, W1, b1, W2, b2,
           edge_index, edge_pairs, path_nodes, path_lengths):
    src = edge_index[0]
    dst = edge_index[1]

    agg = jax.ops.segment_sum(x[src], dst, num_segments=NPAD)
    deg = jax.ops.segment_sum(jnp.ones((E,), jnp.float32), dst, num_segments=NPAD)

    h = _gnn_dense(agg[:, :H], agg[:, H:], deg[:, None],
                   W_gnn, b_gnn.reshape(1, D))

    ar = jnp.arange(L, dtype=path_lengths.dtype)
    nodes_m = jnp.where(ar[None, :] < path_lengths[:, None], path_nodes, N)
    psum = h[nodes_m].sum(axis=1)
    hu = h[edge_pairs[:, 0]]
    hv = h[edge_pairs[:, 1]]

    lens = path_lengths.astype(jnp.float32)[:, None]
    out = _predictor(psum[:, :H], psum[:, H:], hu[:, :H], hu[:, H:],
                     hv[:, :H], hv[:, H:], lens,
                     W_path, b_path.reshape(1, D), W1, b1.reshape(1, D),
                     W2.reshape(1, D), b2.reshape(1, 1))
    return out
